# baseline (speedup 1.0000x reference)
import functools

import jax
import jax.numpy as jnp
from jax.experimental import pallas as pl
from jax.experimental.pallas import tpu as pltpu

_N = 16384
_K = 8192
_C = 32
_BM = 256
_G = _N // _BM


def _vq_body(xf_ref, x2_ref, ytr_ref, y_ref,
             tok_ref, q_ref, loss_ref, perp_ref,
             y2_ref, loss_acc, counts_ref):
    i = pl.program_id(0)

    @pl.when(i == 0)
    def _init():
        yt = ytr_ref[...]
        acc = yt[0:1, :] * yt[0:1, :]
        for c in range(1, _C):
            acc = acc + yt[c:c + 1, :] * yt[c:c + 1, :]
        y2_ref[...] = acc
        loss_acc[...] = jnp.zeros((1, 1), jnp.float32)
        counts_ref[...] = jnp.zeros((1, _K), jnp.float32)

    xf = xf_ref[...]
    xq = xf.astype(jnp.bfloat16).astype(jnp.float32)
    m = jax.lax.dot_general(
        xq, ytr_ref[...], (((1,), (0,)), ((), ())),
        preferred_element_type=jnp.float32,
        precision=jax.lax.Precision.HIGHEST)
    d2 = (x2_ref[...] - 2.0 * m) + y2_ref[...]
    s = jnp.sqrt(jnp.maximum(d2, 0.0))
    minv = jnp.min(s, axis=1, keepdims=True)
    iota = jax.lax.broadcasted_iota(jnp.int32, (1, _K), 1)
    cand = jnp.where(s == minv, iota, jnp.int32(1 << 30))
    tok = jnp.min(cand, axis=1, keepdims=True)
    tok_ref[...] = tok

    oh = (iota == tok).astype(jnp.float32)
    q = jax.lax.dot_general(
        oh, y_ref[...], (((1,), (0,)), ((), ())),
        preferred_element_type=jnp.float32)
    q_ref[...] = q

    r = q - xf
    loss_acc[...] += jnp.sum(r * r).reshape(1, 1)
    ones = jnp.ones((1, _BM), jnp.float32)
    counts_ref[...] += jax.lax.dot_general(
        ones, oh, (((1,), (0,)), ((), ())),
        preferred_element_type=jnp.float32)

    @pl.when(i == _G - 1)
    def _fini():
        loss_ref[...] = loss_acc[...] / jnp.float32(_N * _C)
        p = counts_ref[...] / jnp.float32(_N)
        ent = jnp.sum(p * jnp.log(p + 1e-10))
        perp_ref[...] = jnp.exp(-ent).reshape(1, 1)


@functools.partial(jax.jit, static_argnames=())
def kernel(z, codebook):
    b, c, h, w = z.shape
    flatten = jnp.transpose(z, (0, 2, 3, 1)).reshape(-1, c)
    x2 = jnp.sum(flatten * flatten, axis=1, keepdims=True)
    ytr = codebook.T

    grid = (_G,)
    tok, q, loss, perp = pl.pallas_call(
        _vq_body,
        grid=grid,
        in_specs=[
            pl.BlockSpec((_BM, _C), lambda i: (i, 0)),
            pl.BlockSpec((_BM, 1), lambda i: (i, 0)),
            pl.BlockSpec((_C, _K), lambda i: (0, 0)),
            pl.BlockSpec((_K, _C), lambda i: (0, 0)),
        ],
        out_specs=[
            pl.BlockSpec((_BM, 1), lambda i: (i, 0)),
            pl.BlockSpec((_BM, _C), lambda i: (i, 0)),
            pl.BlockSpec((1, 1), lambda i: (0, 0)),
            pl.BlockSpec((1, 1), lambda i: (0, 0)),
        ],
        out_shape=[
            jax.ShapeDtypeStruct((_N, 1), jnp.int32),
            jax.ShapeDtypeStruct((_N, _C), jnp.float32),
            jax.ShapeDtypeStruct((1, 1), jnp.float32),
            jax.ShapeDtypeStruct((1, 1), jnp.float32),
        ],
        scratch_shapes=[
            pltpu.VMEM((1, _K), jnp.float32),
            pltpu.VMEM((1, 1), jnp.float32),
            pltpu.VMEM((1, _K), jnp.float32),
        ],
    )(flatten, x2, ytr, codebook)

    quantized_st = jnp.transpose(q.reshape(b, h, w, c), (0, 3, 1, 2))
    tokens_out = tok.reshape(b, h, w)
    loss = loss.reshape(())
    perplexity = perp.reshape(())
    return (quantized_st, tokens_out, loss, perplexity)
